# baseline (speedup 1.0000x reference)
import functools

import jax
import jax.numpy as jnp
from jax import lax
from jax.experimental import pallas as pl
from jax.experimental.pallas import tpu as pltpu
from jax.experimental.pallas import tpu_sc as plsc

BATCH = 4096
NSTEP = 11
LENGTH = 20
INPUT_SIZE = 64

B_TC = 1024
B_SC = BATCH - B_TC

NC = 2
NS = 16
NW = NC * NS
B_PER_W = B_SC // NW
GROUP = 16
NGROUP = B_PER_W // GROUP

BLK = 64
NBLK = B_TC // BLK


def _sc_gather(table, steps):
    mesh = plsc.VectorSubcoreMesh(core_axis_name="c", subcore_axis_name="s")

    @functools.partial(
        pl.kernel,
        mesh=mesh,
        out_type=jax.ShapeDtypeStruct((B_SC, LENGTH, INPUT_SIZE),
                                      jnp.float32),
        scratch_types=[
            pltpu.VMEM((B_PER_W,), jnp.int32),
            pltpu.VMEM((GROUP, LENGTH, INPUT_SIZE), jnp.float32),
            pltpu.SemaphoreType.DMA,
        ],
    )
    def k(table_hbm, steps_hbm, out_hbm, steps_v, rows_v, sem):
        wid = lax.axis_index("s") * NC + lax.axis_index("c")
        start = wid * B_PER_W
        pltpu.sync_copy(steps_hbm.at[pl.ds(start, B_PER_W)], steps_v)

        @pl.loop(0, NGROUP)
        def _(g):
            base = g * GROUP
            svec = steps_v[pl.ds(base, GROUP)]
            copies = []
            for j in range(GROUP):
                idx = (start + base + j) * NSTEP + svec[j]
                copies.append(
                    pltpu.make_async_copy(table_hbm.at[idx], rows_v.at[j],
                                          sem))
            for c in copies:
                c.start()
            for c in copies:
                c.wait()
            pltpu.sync_copy(rows_v,
                            out_hbm.at[pl.ds(start + base, GROUP)])

    return k(table, steps)


def _tc_gather(weight, steps):
    def body(s_ref, w_ref, out_ref):
        i = pl.program_id(0)
        base = i * BLK
        for j in range(BLK):
            out_ref[j] = w_ref[j, s_ref[base + j]]

    grid_spec = pltpu.PrefetchScalarGridSpec(
        num_scalar_prefetch=1,
        grid=(NBLK,),
        in_specs=[pl.BlockSpec((BLK, NSTEP, LENGTH, INPUT_SIZE),
                               lambda i, s_ref: (i, 0, 0, 0))],
        out_specs=pl.BlockSpec((BLK, LENGTH, INPUT_SIZE),
                               lambda i, s_ref: (i, 0, 0)),
    )
    return pl.pallas_call(
        body,
        grid_spec=grid_spec,
        out_shape=jax.ShapeDtypeStruct((B_TC, LENGTH, INPUT_SIZE),
                                       jnp.float32),
    )(steps, weight)


def kernel(embeddings, conditions, steps, weight, lengths):
    steps32 = steps.astype(jnp.int32)
    out_tc = _tc_gather(weight, steps32[:B_TC])
    table_sc = weight[B_TC:].reshape(B_SC * NSTEP, LENGTH, INPUT_SIZE)
    out_sc = _sc_gather(table_sc, steps32[B_TC:])
    out = jnp.concatenate([out_tc, out_sc], axis=0)
    return (out, lengths)


# speedup vs baseline: 2.8531x; 2.8531x over previous
import functools

import jax
import jax.numpy as jnp
from jax import lax
from jax.experimental import pallas as pl
from jax.experimental.pallas import tpu as pltpu
from jax.experimental.pallas import tpu_sc as plsc

BATCH = 4096
NSTEP = 11
LENGTH = 20
INPUT_SIZE = 64

NC = 2
NS = 16
NW = NC * NS
B_PER_W = BATCH // NW
GROUP = 16
NGROUP = B_PER_W // GROUP


def _gather_rows(table, steps):
    mesh = plsc.VectorSubcoreMesh(core_axis_name="c", subcore_axis_name="s")

    @functools.partial(
        pl.kernel,
        mesh=mesh,
        out_type=jax.ShapeDtypeStruct((BATCH, LENGTH, INPUT_SIZE),
                                      jnp.float32),
        scratch_types=[
            pltpu.VMEM((B_PER_W,), jnp.int32),
            pltpu.VMEM((GROUP, LENGTH, INPUT_SIZE), jnp.float32),
            pltpu.VMEM((GROUP, LENGTH, INPUT_SIZE), jnp.float32),
            pltpu.SemaphoreType.DMA,
            pltpu.SemaphoreType.DMA,
        ],
    )
    def k(table_hbm, steps_hbm, out_hbm, steps_v, rows_a, rows_b, sem_g,
          sem_w):
        wid = lax.axis_index("s") * NC + lax.axis_index("c")
        start = wid * B_PER_W
        pltpu.sync_copy(steps_hbm.at[pl.ds(start, B_PER_W)], steps_v)

        bufs = (rows_a, rows_b)

        def fire_gather(g):
            base = g * GROUP
            svec = steps_v[pl.ds(base, GROUP)]
            buf = bufs[g % 2]
            copies = []
            for j in range(GROUP):
                idx = (start + base + j) * NSTEP + svec[j]
                copies.append(
                    pltpu.make_async_copy(table_hbm.at[idx], buf.at[j],
                                          sem_g))
            for c in copies:
                c.start()
            return copies

        wb = [None] * NGROUP
        pending = fire_gather(0)
        for g in range(NGROUP):
            if g + 1 < NGROUP:
                if g - 1 >= 0:
                    wb[g - 1].wait()
                nxt = fire_gather(g + 1)
            for c in pending:
                c.wait()
            wb[g] = pltpu.make_async_copy(
                bufs[g % 2], out_hbm.at[pl.ds(start + g * GROUP, GROUP)],
                sem_w)
            wb[g].start()
            if g + 1 < NGROUP:
                pending = nxt
        wb[NGROUP - 2].wait()
        wb[NGROUP - 1].wait()

    return k(table, steps)


def kernel(embeddings, conditions, steps, weight, lengths):
    table = weight.reshape(BATCH * NSTEP, LENGTH, INPUT_SIZE)
    out = _gather_rows(table, steps.astype(jnp.int32))
    return (out, lengths)
